# Initial kernel scaffold; baseline (speedup 1.0000x reference)
import functools

import jax
import jax.numpy as jnp
from jax import lax
from jax.experimental import pallas as pl
from jax.experimental.pallas import tpu as pltpu
from jax.experimental.pallas import tpu_sc as plsc

N = 10000
NP = 10240
E = 320000
NC, NS = 2, 16
NW = NC * NS
EPT = E // NW
K = 400
NCH = EPT // K
STRIPE = NP // NS

R = 256
G = NP // R

_DIMS = [128, 80, 160, 112, 160, 176, 96, 144, 96, 128, 96, 160]


def _leaky(v):
    return jnp.where(v >= 0, v, 0.01 * v)


_MESH = plsc.VectorSubcoreMesh(core_axis_name="c", subcore_axis_name="s")


@functools.partial(
    pl.kernel,
    out_type=jax.ShapeDtypeStruct((NC, 2, NP), jnp.float32),
    mesh=_MESH,
    scratch_types=[
        pltpu.VMEM((K,), jnp.int32),
        pltpu.VMEM((K,), jnp.int32),
        pltpu.VMEM((K,), jnp.float32),
        pltpu.VMEM_SHARED((NP,), jnp.float32),
        pltpu.VMEM_SHARED((NP,), jnp.float32),
    ],
)
def _deg_kernel(src_hbm, dst_hbm, zeros_hbm, ones_hbm, out_hbm,
                sidx, didx, ones_v, cnt_s, cnt_d):
    cid = lax.axis_index("c")
    sid = lax.axis_index("s")
    wid = sid * NC + cid
    r0 = sid * STRIPE
    pltpu.sync_copy(zeros_hbm.at[pl.ds(r0, STRIPE)], cnt_s.at[pl.ds(r0, STRIPE)])
    pltpu.sync_copy(zeros_hbm.at[pl.ds(r0, STRIPE)], cnt_d.at[pl.ds(r0, STRIPE)])
    pltpu.sync_copy(ones_hbm, ones_v)
    plsc.subcore_barrier()
    base = wid * EPT

    def body(c, carry):
        off = base + c * K
        pltpu.sync_copy(src_hbm.at[pl.ds(off, K)], sidx)
        pltpu.sync_copy(dst_hbm.at[pl.ds(off, K)], didx)
        pltpu.sync_copy(ones_v, cnt_s.at[sidx], add=True)
        pltpu.sync_copy(ones_v, cnt_d.at[didx], add=True)
        return carry

    lax.fori_loop(0, NCH, body, 0, unroll=False)
    plsc.subcore_barrier()
    pltpu.sync_copy(cnt_s.at[pl.ds(r0, STRIPE)], out_hbm.at[cid, 0, pl.ds(r0, STRIPE)])
    pltpu.sync_copy(cnt_d.at[pl.ds(r0, STRIPE)], out_hbm.at[cid, 1, pl.ds(r0, STRIPE)])


@functools.cache
def _make_prop(w):
    @functools.partial(
        pl.kernel,
        out_type=jax.ShapeDtypeStruct((NC, NP, w), jnp.float32),
        mesh=_MESH,
        scratch_types=[
            pltpu.VMEM((K,), jnp.int32),
            pltpu.VMEM((K,), jnp.int32),
            pltpu.VMEM((K, w), jnp.float32),
            pltpu.VMEM_SHARED((NP, w), jnp.float32),
            pltpu.SemaphoreType.DMA,
        ],
    )
    def prop(t_hbm, src_hbm, dst_hbm, zeros_hbm, out_hbm,
             sidx, didx, rows, agg, sem):
        cid = lax.axis_index("c")
        sid = lax.axis_index("s")
        wid = sid * NC + cid
        r0 = sid * STRIPE
        pltpu.sync_copy(zeros_hbm.at[pl.ds(r0, STRIPE)], agg.at[pl.ds(r0, STRIPE)])
        plsc.subcore_barrier()
        base = wid * EPT

        def body(c, carry):
            off = base + c * K
            pltpu.sync_copy(src_hbm.at[pl.ds(off, K)], sidx)
            pltpu.sync_copy(dst_hbm.at[pl.ds(off, K)], didx)
            pltpu.async_copy(t_hbm.at[sidx], rows, sem).wait()
            pltpu.sync_copy(rows, agg.at[didx], add=True)
            return carry

        lax.fori_loop(0, NCH, body, 0, unroll=False)
        plsc.subcore_barrier()
        pltpu.sync_copy(agg.at[pl.ds(r0, STRIPE)],
                        out_hbm.at[cid, pl.ds(r0, STRIPE)])

    return prop


def _scales(degp):
    def body(dr, sr):
        d = dr[...]
        sr[...] = lax.rsqrt(jnp.maximum(d[:, 0, :] + d[:, 1, :], 1.0))

    return pl.pallas_call(
        body,
        grid=(1,),
        in_specs=[pl.BlockSpec((NC, 2, NP), lambda i: (0, 0, 0))],
        out_specs=pl.BlockSpec((2, NP), lambda i: (0, 0)),
        out_shape=jax.ShapeDtypeStruct((2, NP), jnp.float32),
    )(degp)


def _dense_stage(parts, s_in, b, Wpost, Wpre, s_out):
    win = parts.shape[2]
    wmid = b.shape[1]
    wout = Wpre.shape[1] if Wpre is not None else wmid

    args = [parts, parts, s_in, b]
    specs = [
        pl.BlockSpec((1, R, win), lambda i: (0, i, 0)),
        pl.BlockSpec((1, R, win), lambda i: (1, i, 0)),
        pl.BlockSpec((R, 1), lambda i: (i, 0)),
        pl.BlockSpec((1, wmid), lambda i: (0, 0)),
    ]
    if Wpost is not None:
        args.append(Wpost)
        specs.append(pl.BlockSpec(Wpost.shape, lambda i: (0, 0)))
    if Wpre is not None:
        args.append(Wpre)
        specs.append(pl.BlockSpec(Wpre.shape, lambda i: (0, 0)))
    args.append(s_out)
    specs.append(pl.BlockSpec((R, 1), lambda i: (i, 0)))

    def body(*refs):
        it = iter(refs)
        a0r, a1r, sir, br = next(it), next(it), next(it), next(it)
        Wpo = next(it) if Wpost is not None else None
        Wpr = next(it) if Wpre is not None else None
        sor, outr = next(it), next(it)
        a = (a0r[0] + a1r[0]) * sir[...]
        if Wpo is not None:
            a = jnp.dot(a, Wpo[...], preferred_element_type=jnp.float32)
        a = _leaky(a + br[...])
        if Wpr is not None:
            a = jnp.dot(a, Wpr[...], preferred_element_type=jnp.float32)
        outr[...] = a * sor[...]

    return pl.pallas_call(
        body,
        grid=(G,),
        in_specs=specs,
        out_specs=pl.BlockSpec((R, wout), lambda i: (i, 0)),
        out_shape=jax.ShapeDtypeStruct((NP, wout), jnp.float32),
    )(*args)


def _first_stage(xp, W1, s_out):
    win, wout = W1.shape

    def body(xr, Wr, sor, outr):
        a = jnp.dot(xr[...], Wr[...], preferred_element_type=jnp.float32)
        outr[...] = a * sor[...]

    return pl.pallas_call(
        body,
        grid=(G,),
        in_specs=[
            pl.BlockSpec((R, win), lambda i: (i, 0)),
            pl.BlockSpec((win, wout), lambda i: (0, 0)),
            pl.BlockSpec((R, 1), lambda i: (i, 0)),
        ],
        out_specs=pl.BlockSpec((R, wout), lambda i: (i, 0)),
        out_shape=jax.ShapeDtypeStruct((NP, wout), jnp.float32),
    )(xp, W1, s_out)


def _head(parts, s_in, W11, b11, Wd1, bd1, Wd2, bd2):
    win = parts.shape[2]
    wmid = W11.shape[1]

    def body(a0r, a1r, sir, W11r, b11r, Wd1r, bd1r, Wd2r, bd2r, outr, acc):
        i = pl.program_id(0)
        a = (a0r[0] + a1r[0]) * sir[...]
        u = _leaky(jnp.dot(a, W11r[...], preferred_element_type=jnp.float32)
                   + b11r[...])
        rows = i * R + lax.broadcasted_iota(jnp.int32, (R, 1), 0)
        u = jnp.where(rows < N, u, 0.0)

        @pl.when(i == 0)
        def _():
            acc[...] = jnp.zeros_like(acc)

        acc[...] += jnp.sum(u, axis=0, keepdims=True)

        @pl.when(i == G - 1)
        def _():
            m = _leaky(acc[...] * (1.0 / N))
            h1 = _leaky(jnp.dot(m, Wd1r[...], preferred_element_type=jnp.float32)
                        + bd1r[...])
            h2 = (jnp.dot(h1, Wd2r[...], preferred_element_type=jnp.float32)
                  + bd2r[...])
            outr[...] = jax.nn.sigmoid(h2)

    return pl.pallas_call(
        body,
        grid=(G,),
        in_specs=[
            pl.BlockSpec((1, R, win), lambda i: (0, i, 0)),
            pl.BlockSpec((1, R, win), lambda i: (1, i, 0)),
            pl.BlockSpec((R, 1), lambda i: (i, 0)),
            pl.BlockSpec(W11.shape, lambda i: (0, 0)),
            pl.BlockSpec((1, wmid), lambda i: (0, 0)),
            pl.BlockSpec(Wd1.shape, lambda i: (0, 0)),
            pl.BlockSpec((1, Wd1.shape[1]), lambda i: (0, 0)),
            pl.BlockSpec(Wd2.shape, lambda i: (0, 0)),
            pl.BlockSpec((1, Wd2.shape[1]), lambda i: (0, 0)),
        ],
        out_specs=pl.BlockSpec((1, Wd2.shape[1]), lambda i: (0, 0)),
        out_shape=jax.ShapeDtypeStruct((1, Wd2.shape[1]), jnp.float32),
        scratch_shapes=[pltpu.VMEM((1, wmid), jnp.float32)],
    )(parts, parts, s_in, W11, b11, Wd1, bd1, Wd2, bd2)


def kernel(x, edge_index, edge_feat, params):
    del edge_feat
    src = edge_index[0]
    dst = edge_index[1]
    f32 = jnp.float32

    xp = jnp.concatenate([x, jnp.zeros((NP - N, x.shape[1]), f32)], axis=0)
    zeros1 = jnp.zeros((NP,), f32)
    ones_k = jnp.ones((K,), f32)

    degp = _deg_kernel(src, dst, zeros1, ones_k)
    s2 = _scales(degp)
    s_out = s2[0].reshape(NP, 1)
    s_in = s2[1].reshape(NP, 1)

    Ws = [params['W%d' % (i + 1)] for i in range(11)]
    bs = [params['b%d' % (i + 1)].reshape(1, -1) for i in range(11)]
    mf = [_DIMS[i + 1] < _DIMS[i] for i in range(11)]

    t = _first_stage(xp, Ws[0], s_out)
    out = None
    for i in range(11):
        w = t.shape[1]
        zeros_w = jnp.zeros((NP, w), f32)
        parts = _make_prop(w)(t, src, dst, zeros_w)
        if i < 10:
            Wpost = None if mf[i] else Ws[i]
            Wpre = Ws[i + 1] if mf[i + 1] else None
            t = _dense_stage(parts, s_in, bs[i], Wpost, Wpre, s_out)
        else:
            out = _head(parts, s_in, Ws[10], bs[10],
                        params['Wd1'], params['bd1'].reshape(1, -1),
                        params['Wd2'], params['bd2'].reshape(1, -1))
    return out


# speedup vs baseline: 7.8468x; 7.8468x over previous
import functools

import jax
import jax.numpy as jnp
from jax import lax
from jax.experimental import pallas as pl
from jax.experimental.pallas import tpu as pltpu
from jax.experimental.pallas import tpu_sc as plsc

N = 10000
NP = 10240
E = 320000
NC, NS = 2, 16
NW = NC * NS
EPT = E // NW
K = 400
NCH = EPT // K
STRIPE = NP // NS

R = 256
G = NP // R

_DIMS = [128, 80, 160, 112, 160, 176, 96, 144, 96, 128, 96, 160]


def _leaky(v):
    return jnp.where(v >= 0, v, 0.01 * v)


@functools.cache
def _mesh():
    return plsc.VectorSubcoreMesh(core_axis_name="c", subcore_axis_name="s",
                                  num_cores=NC, num_subcores=NS)


@functools.cache
def _make_deg():
    @functools.partial(
        pl.kernel,
        out_type=jax.ShapeDtypeStruct((NC, 2, NP), jnp.float32),
        mesh=_mesh(),
        scratch_types=[
            pltpu.VMEM((K,), jnp.int32),
            pltpu.VMEM((K,), jnp.int32),
            pltpu.VMEM((K,), jnp.float32),
            pltpu.VMEM_SHARED((NP,), jnp.float32),
            pltpu.VMEM_SHARED((NP,), jnp.float32),
        ],
    )
    def _deg_kernel(src_hbm, dst_hbm, zeros_hbm, ones_hbm, out_hbm,
                    sidx, didx, ones_v, cnt_s, cnt_d):
        cid = lax.axis_index("c")
        sid = lax.axis_index("s")
        wid = sid * NC + cid
        r0 = sid * STRIPE
        pltpu.sync_copy(zeros_hbm.at[pl.ds(r0, STRIPE)], cnt_s.at[pl.ds(r0, STRIPE)])
        pltpu.sync_copy(zeros_hbm.at[pl.ds(r0, STRIPE)], cnt_d.at[pl.ds(r0, STRIPE)])
        pltpu.sync_copy(ones_hbm, ones_v)
        plsc.subcore_barrier()
        base = wid * EPT

        def body(c, carry):
            off = base + c * K
            pltpu.sync_copy(src_hbm.at[pl.ds(off, K)], sidx)
            pltpu.sync_copy(dst_hbm.at[pl.ds(off, K)], didx)
            pltpu.sync_copy(ones_v, cnt_s.at[sidx], add=True)
            pltpu.sync_copy(ones_v, cnt_d.at[didx], add=True)
            return carry

        lax.fori_loop(0, NCH, body, 0, unroll=False)
        plsc.subcore_barrier()
        pltpu.sync_copy(cnt_s.at[pl.ds(r0, STRIPE)],
                        out_hbm.at[cid, 0, pl.ds(r0, STRIPE)])
        pltpu.sync_copy(cnt_d.at[pl.ds(r0, STRIPE)],
                        out_hbm.at[cid, 1, pl.ds(r0, STRIPE)])

    return _deg_kernel


PW = 112


@functools.cache
def _make_prop():
    w = PW
    @functools.partial(
        pl.kernel,
        out_type=jax.ShapeDtypeStruct((NC, NP, w), jnp.float32),
        mesh=_mesh(),
        scratch_types=[
            pltpu.VMEM((K,), jnp.int32),
            pltpu.VMEM((K,), jnp.int32),
            pltpu.VMEM((K, w), jnp.float32),
            pltpu.VMEM_SHARED((NP, w), jnp.float32),
            pltpu.SemaphoreType.DMA,
        ],
        compiler_params=pltpu.CompilerParams(use_tc_tiling_on_sc=False),
    )
    def prop(t_hbm, src_hbm, dst_hbm, zeros_hbm, out_hbm,
             sidx, didx, rows, agg, sem):
        cid = lax.axis_index("c")
        sid = lax.axis_index("s")
        wid = sid * NC + cid
        r0 = sid * STRIPE
        pltpu.sync_copy(zeros_hbm.at[pl.ds(r0, STRIPE)], agg.at[pl.ds(r0, STRIPE)])
        plsc.subcore_barrier()
        base = wid * EPT

        def body(c, carry):
            off = base + c * K
            pltpu.sync_copy(src_hbm.at[pl.ds(off, K)], sidx)
            pltpu.sync_copy(dst_hbm.at[pl.ds(off, K)], didx)
            pltpu.async_copy(t_hbm.at[sidx], rows, sem).wait()
            pltpu.sync_copy(rows, agg.at[didx], add=True)
            return carry

        lax.fori_loop(0, NCH, body, 0, unroll=False)
        plsc.subcore_barrier()
        pltpu.sync_copy(agg.at[pl.ds(r0, STRIPE)],
                        out_hbm.at[cid, pl.ds(r0, STRIPE)])

    return prop


def _scales(degp):
    def body(dr, sr):
        d = dr[...]
        sr[...] = lax.rsqrt(jnp.maximum(d[0] + d[1], 1.0))

    return pl.pallas_call(
        body,
        grid=(1,),
        in_specs=[pl.BlockSpec((NC, 2, NP), lambda i: (0, 0, 0))],
        out_specs=pl.BlockSpec((2, NP), lambda i: (0, 0)),
        out_shape=jax.ShapeDtypeStruct((2, NP), jnp.float32),
    )(degp)


def _dense_stage(parts, s_in, b, Wpost, Wpre, s_out):
    win = parts.shape[2]
    wmid = b.shape[1]
    wout = Wpre.shape[1] if Wpre is not None else wmid

    args = [parts, parts, s_in, b]
    specs = [
        pl.BlockSpec((1, R, win), lambda i: (0, i, 0)),
        pl.BlockSpec((1, R, win), lambda i: (1, i, 0)),
        pl.BlockSpec((R, 1), lambda i: (i, 0)),
        pl.BlockSpec((1, wmid), lambda i: (0, 0)),
    ]
    if Wpost is not None:
        args.append(Wpost)
        specs.append(pl.BlockSpec(Wpost.shape, lambda i: (0, 0)))
    if Wpre is not None:
        args.append(Wpre)
        specs.append(pl.BlockSpec(Wpre.shape, lambda i: (0, 0)))
    args.append(s_out)
    specs.append(pl.BlockSpec((R, 1), lambda i: (i, 0)))

    def body(*refs):
        it = iter(refs)
        a0r, a1r, sir, br = next(it), next(it), next(it), next(it)
        Wpo = next(it) if Wpost is not None else None
        Wpr = next(it) if Wpre is not None else None
        sor, outr = next(it), next(it)
        a = (a0r[0] + a1r[0]) * sir[...]
        if Wpo is not None:
            a = jnp.dot(a, Wpo[...], preferred_element_type=jnp.float32)
        a = _leaky(a + br[...])
        if Wpr is not None:
            a = jnp.dot(a, Wpr[...], preferred_element_type=jnp.float32)
        outr[...] = a * sor[...]

    return pl.pallas_call(
        body,
        grid=(G,),
        in_specs=specs,
        out_specs=pl.BlockSpec((R, wout), lambda i: (i, 0)),
        out_shape=jax.ShapeDtypeStruct((NP, wout), jnp.float32),
    )(*args)


def _first_stage(xp, W1, s_out):
    win, wout = W1.shape

    def body(xr, Wr, sor, outr):
        a = jnp.dot(xr[...], Wr[...], preferred_element_type=jnp.float32)
        outr[...] = a * sor[...]

    return pl.pallas_call(
        body,
        grid=(G,),
        in_specs=[
            pl.BlockSpec((R, win), lambda i: (i, 0)),
            pl.BlockSpec((win, wout), lambda i: (0, 0)),
            pl.BlockSpec((R, 1), lambda i: (i, 0)),
        ],
        out_specs=pl.BlockSpec((R, wout), lambda i: (i, 0)),
        out_shape=jax.ShapeDtypeStruct((NP, wout), jnp.float32),
    )(xp, W1, s_out)


def _head(parts, s_in, W11, b11, Wd1, bd1, Wd2, bd2):
    win = parts.shape[2]
    wmid = W11.shape[1]

    def body(a0r, a1r, sir, W11r, b11r, Wd1r, bd1r, Wd2r, bd2r, outr, acc):
        i = pl.program_id(0)
        a = (a0r[0] + a1r[0]) * sir[...]
        u = _leaky(jnp.dot(a, W11r[...], preferred_element_type=jnp.float32)
                   + b11r[...])
        rows = i * R + lax.broadcasted_iota(jnp.int32, (R, 1), 0)
        u = jnp.where(rows < N, u, 0.0)

        @pl.when(i == 0)
        def _():
            acc[...] = jnp.zeros_like(acc)

        acc[...] += jnp.sum(u, axis=0, keepdims=True)

        @pl.when(i == G - 1)
        def _():
            m = _leaky(acc[...] * (1.0 / N))
            h1 = _leaky(jnp.dot(m, Wd1r[...], preferred_element_type=jnp.float32)
                        + bd1r[...])
            h2 = (jnp.dot(h1, Wd2r[...], preferred_element_type=jnp.float32)
                  + bd2r[...])
            outr[...] = jax.nn.sigmoid(h2)

    return pl.pallas_call(
        body,
        grid=(G,),
        in_specs=[
            pl.BlockSpec((1, R, win), lambda i: (0, i, 0)),
            pl.BlockSpec((1, R, win), lambda i: (1, i, 0)),
            pl.BlockSpec((R, 1), lambda i: (i, 0)),
            pl.BlockSpec(W11.shape, lambda i: (0, 0)),
            pl.BlockSpec((1, wmid), lambda i: (0, 0)),
            pl.BlockSpec(Wd1.shape, lambda i: (0, 0)),
            pl.BlockSpec((1, Wd1.shape[1]), lambda i: (0, 0)),
            pl.BlockSpec(Wd2.shape, lambda i: (0, 0)),
            pl.BlockSpec((1, Wd2.shape[1]), lambda i: (0, 0)),
        ],
        out_specs=pl.BlockSpec((1, Wd2.shape[1]), lambda i: (0, 0)),
        out_shape=jax.ShapeDtypeStruct((1, Wd2.shape[1]), jnp.float32),
        scratch_shapes=[pltpu.VMEM((1, wmid), jnp.float32)],
    )(parts, parts, s_in, W11, b11, Wd1, bd1, Wd2, bd2)


def kernel(x, edge_index, edge_feat, params):
    del edge_feat
    src = edge_index[0]
    dst = edge_index[1]
    f32 = jnp.float32

    xp = jnp.concatenate([x, jnp.zeros((NP - N, x.shape[1]), f32)], axis=0)
    zeros1 = jnp.zeros((NP,), f32)
    ones_k = jnp.ones((K,), f32)

    degp = _make_deg()(src, dst, zeros1, ones_k)
    s2 = _scales(degp)
    s_out = s2[0].reshape(NP, 1)
    s_in = s2[1].reshape(NP, 1)

    Ws = [params['W%d' % (i + 1)] for i in range(11)]
    bs = [params['b%d' % (i + 1)].reshape(1, -1) for i in range(11)]
    mf = [_DIMS[i + 1] < _DIMS[i] for i in range(11)]

    zeros_pw = jnp.zeros((NP, PW), f32)

    def _propagate(t):
        w = t.shape[1]
        outs = []
        for c0 in range(0, w, PW):
            cw = min(PW, w - c0)
            tck = t[:, c0:c0 + cw]
            if cw < PW:
                tck = jnp.pad(tck, ((0, 0), (0, PW - cw)))
            p = _make_prop()(tck, src, dst, zeros_pw)
            outs.append(p[:, :, :cw])
        return outs[0] if len(outs) == 1 else jnp.concatenate(outs, axis=2)

    t = _first_stage(xp, Ws[0], s_out)
    out = None
    for i in range(11):
        parts = _propagate(t)
        if i < 10:
            Wpost = None if mf[i] else Ws[i]
            Wpre = Ws[i + 1] if mf[i + 1] else None
            t = _dense_stage(parts, s_in, bs[i], Wpost, Wpre, s_out)
        else:
            out = _head(parts, s_in, Ws[10], bs[10],
                        params['Wd1'], params['bd1'].reshape(1, -1),
                        params['Wd2'], params['bd2'].reshape(1, -1))
    return out
